# Initial kernel scaffold; baseline (speedup 1.0000x reference)
import jax
import jax.numpy as jnp
from jax.experimental import pallas as pl


def kernel(neuromod_state, codebook):
    raise NotImplementedError("write your pallas kernel here")


# speedup vs baseline: 2.5835x; 2.5835x over previous
import functools
import jax
import jax.numpy as jnp
from jax import lax
from jax.experimental import pallas as pl
from jax.experimental.pallas import tpu as pltpu

_N_MOODS = 16
_D = 5


def _vq_block(z_ref, cb_ref, zq_ref, idx_ref, s1_ref, s2_ref):
    i = pl.program_id(0)
    z = z_ref[...]
    cb = cb_ref[...]

    zsq = jnp.sum(z * z, axis=1, keepdims=True)
    csq = jnp.sum(cb * cb, axis=1, keepdims=True).T
    cross = jax.lax.dot_general(
        z, cb, (((1,), (1,)), ((), ())),
        preferred_element_type=jnp.float32)
    d2 = zsq - 2.0 * cross + csq

    m = jnp.min(d2, axis=1, keepdims=True)
    ii = lax.broadcasted_iota(jnp.int32, d2.shape, 1)
    eq = d2 == m
    idx2d = jnp.min(jnp.where(eq, ii, _N_MOODS), axis=1, keepdims=True)

    oh = (ii == idx2d).astype(jnp.float32)
    codes = jax.lax.dot_general(
        oh, cb, (((1,), (0,)), ((), ())),
        preferred_element_type=jnp.float32)

    zq_ref[...] = z + (codes - z)
    idx_ref[...] = idx2d

    min_d2 = jnp.maximum(m, 0.0)
    s1 = jnp.sum(min_d2, keepdims=True).reshape(1, 1)
    s2 = jnp.sum(jnp.sqrt(min_d2), keepdims=True).reshape(1, 1)

    @pl.when(i == 0)
    def _init():
        s1_ref[...] = jnp.zeros((1, 1), jnp.float32)
        s2_ref[...] = jnp.zeros((1, 1), jnp.float32)

    s1_ref[...] += s1
    s2_ref[...] += s2


@functools.partial(jax.jit, static_argnames=())
def kernel(neuromod_state, codebook):
    z = neuromod_state
    if z.ndim == 1:
        z = z[None, :]
    b, d = z.shape
    blk = 8192
    if b % blk != 0:
        blk = b
    grid = (b // blk,)

    zq, idx2d, s1, s2 = pl.pallas_call(
        _vq_block,
        grid=grid,
        in_specs=[
            pl.BlockSpec((blk, d), lambda i: (i, 0)),
            pl.BlockSpec((_N_MOODS, d), lambda i: (0, 0)),
        ],
        out_specs=[
            pl.BlockSpec((blk, d), lambda i: (i, 0)),
            pl.BlockSpec((blk, 1), lambda i: (i, 0)),
            pl.BlockSpec((1, 1), lambda i: (0, 0)),
            pl.BlockSpec((1, 1), lambda i: (0, 0)),
        ],
        out_shape=[
            jax.ShapeDtypeStruct((b, d), jnp.float32),
            jax.ShapeDtypeStruct((b, 1), jnp.int32),
            jax.ShapeDtypeStruct((1, 1), jnp.float32),
            jax.ShapeDtypeStruct((1, 1), jnp.float32),
        ],
        compiler_params=pltpu.CompilerParams(
            dimension_semantics=("arbitrary",),
        ),
    )(z, codebook)

    commit_loss = (2.0 / (b * d)) * s1[0, 0]
    mean_dist = s2[0, 0] / b
    return zq, idx2d[:, 0], commit_loss, mean_dist


# speedup vs baseline: 3.3248x; 1.2869x over previous
import functools
import numpy as np
import jax
import jax.numpy as jnp
from jax import lax
from jax.experimental import pallas as pl
from jax.experimental.pallas import tpu as pltpu

_K = 16
_D = 5
_P = 128
_W = _D * _P


def _vq_body(cb_ref, csq_ref, sel_ref, selt_ref, z_ref, zq_ref, idx_ref,
             s1_ref, s2_ref):
    i = pl.program_id(0)
    z640 = z_ref[...]
    zd = jax.lax.dot_general(
        z640, sel_ref[...], (((1,), (0,)), ((), ())),
        preferred_element_type=jnp.float32)
    zs = [zd[:, d * _P:(d + 1) * _P] for d in range(_D)]

    zsq = zs[0] * zs[0]
    for d in range(1, _D):
        zsq += zs[d] * zs[d]

    best = None
    bidx = None
    for k in range(_K):
        score = jnp.full_like(zsq, csq_ref[k, 0])
        for d in range(_D):
            score += zs[d] * (-2.0 * cb_ref[k, d])
        if best is None:
            best = score
            bidx = jnp.zeros(score.shape, jnp.int32)
        else:
            m = score < best
            best = jnp.where(m, score, best)
            bidx = jnp.where(m, k, bidx)

    min_d2 = jnp.maximum(best + zsq, 0.0)
    s1 = jnp.sum(min_d2, keepdims=True).reshape(1, 1)
    s2 = jnp.sum(jnp.sqrt(min_d2), keepdims=True).reshape(1, 1)

    cds = []
    for d in range(_D):
        c = jnp.full(bidx.shape, cb_ref[0, d], jnp.float32)
        for k in range(1, _K):
            c = jnp.where(bidx == k, cb_ref[k, d], c)
        cds.append(c)
    codes = jnp.concatenate(cds, axis=1)
    codes640 = jax.lax.dot_general(
        codes, selt_ref[...], (((1,), (0,)), ((), ())),
        preferred_element_type=jnp.float32)

    zq_ref[...] = z640 + (codes640 - z640)
    idx_ref[...] = bidx

    @pl.when(i == 0)
    def _init():
        s1_ref[...] = jnp.zeros((1, 1), jnp.float32)
        s2_ref[...] = jnp.zeros((1, 1), jnp.float32)

    s1_ref[...] += s1
    s2_ref[...] += s2


def _make_sel():
    p = np.arange(_P)
    s = np.zeros((_W, _W), np.float32)
    for d in range(_D):
        s[_D * p + d, d * _P + p] = 1.0
    return jnp.asarray(s), jnp.asarray(s.T)


@jax.jit
def kernel(neuromod_state, codebook):
    z = neuromod_state
    if z.ndim == 1:
        z = z[None, :]
    b, d = z.shape
    assert d == _D and b % _P == 0
    rows = b // _P
    rb = 256
    while rows % rb != 0:
        rb //= 2
    grid = (rows // rb,)

    z640 = z.reshape(rows, _W)
    sel, selt = _make_sel()
    csq = jnp.sum(codebook * codebook, axis=1, keepdims=True)

    zq640, idx2d, s1, s2 = pl.pallas_call(
        _vq_body,
        grid=grid,
        in_specs=[
            pl.BlockSpec(memory_space=pltpu.SMEM),
            pl.BlockSpec(memory_space=pltpu.SMEM),
            pl.BlockSpec((_W, _W), lambda i: (0, 0)),
            pl.BlockSpec((_W, _W), lambda i: (0, 0)),
            pl.BlockSpec((rb, _W), lambda i: (i, 0)),
        ],
        out_specs=[
            pl.BlockSpec((rb, _W), lambda i: (i, 0)),
            pl.BlockSpec((rb, _P), lambda i: (i, 0)),
            pl.BlockSpec((1, 1), lambda i: (0, 0)),
            pl.BlockSpec((1, 1), lambda i: (0, 0)),
        ],
        out_shape=[
            jax.ShapeDtypeStruct((rows, _W), jnp.float32),
            jax.ShapeDtypeStruct((rows, _P), jnp.int32),
            jax.ShapeDtypeStruct((1, 1), jnp.float32),
            jax.ShapeDtypeStruct((1, 1), jnp.float32),
        ],
        compiler_params=pltpu.CompilerParams(
            dimension_semantics=("arbitrary",),
        ),
    )(codebook, csq, sel, selt, z640)

    commit_loss = (2.0 / (b * d)) * s1[0, 0]
    mean_dist = s2[0, 0] / b
    return (zq640.reshape(b, d), idx2d.reshape(b), commit_loss, mean_dist)


# speedup vs baseline: 4.2798x; 1.2873x over previous
import numpy as np
import jax
import jax.numpy as jnp
from jax import lax
from jax.experimental import pallas as pl
from jax.experimental.pallas import tpu as pltpu

_K = 16
_D = 5


def _vq_body(cb_ref, cb2_ref, csq_ref, z_ref, zq_ref, idx_ref, s1_ref, s2_ref):
    i = pl.program_id(0)
    z = z_ref[...]
    blk = z.shape[0]

    crossT = jax.lax.dot_general(
        cb2_ref[...], z, (((1,), (1,)), ((), ())),
        preferred_element_type=jnp.float32)
    scoreT = crossT + csq_ref[...]

    zz = z * z
    zsqT = jax.lax.dot_general(
        jnp.ones((1, _D), jnp.float32), zz, (((1,), (1,)), ((), ())),
        preferred_element_type=jnp.float32)

    mnT = jnp.min(scoreT, axis=0, keepdims=True)
    iiT = lax.broadcasted_iota(jnp.int32, scoreT.shape, 0)
    idxT = jnp.min(jnp.where(scoreT == mnT, iiT, _K), axis=0, keepdims=True)
    ohT = (iiT == idxT).astype(jnp.float32)

    codes = jax.lax.dot_general(
        ohT, cb_ref[...], (((0,), (0,)), ((), ())),
        preferred_element_type=jnp.float32)
    zq_ref[...] = z + (codes - z)
    idx_ref[...] = idxT.reshape(blk)

    min_d2 = jnp.maximum(mnT + zsqT, 0.0)
    s1 = jnp.sum(min_d2, keepdims=True).reshape(1, 1)
    s2 = jnp.sum(jnp.sqrt(min_d2), keepdims=True).reshape(1, 1)

    @pl.when(i == 0)
    def _init():
        s1_ref[...] = jnp.zeros((1, 1), jnp.float32)
        s2_ref[...] = jnp.zeros((1, 1), jnp.float32)

    s1_ref[...] += s1
    s2_ref[...] += s2


@jax.jit
def kernel(neuromod_state, codebook):
    z = neuromod_state
    if z.ndim == 1:
        z = z[None, :]
    b, d = z.shape
    blk = 8192
    while b % blk != 0:
        blk //= 2
    grid = (b // blk,)

    cb2 = -2.0 * codebook
    csq = jnp.sum(codebook * codebook, axis=1, keepdims=True)

    zq, idx, s1, s2 = pl.pallas_call(
        _vq_body,
        grid=grid,
        in_specs=[
            pl.BlockSpec((_K, d), lambda i: (0, 0)),
            pl.BlockSpec((_K, d), lambda i: (0, 0)),
            pl.BlockSpec((_K, 1), lambda i: (0, 0)),
            pl.BlockSpec((blk, d), lambda i: (i, 0)),
        ],
        out_specs=[
            pl.BlockSpec((blk, d), lambda i: (i, 0)),
            pl.BlockSpec((blk,), lambda i: (i,)),
            pl.BlockSpec((1, 1), lambda i: (0, 0)),
            pl.BlockSpec((1, 1), lambda i: (0, 0)),
        ],
        out_shape=[
            jax.ShapeDtypeStruct((b, d), jnp.float32),
            jax.ShapeDtypeStruct((b,), jnp.int32),
            jax.ShapeDtypeStruct((1, 1), jnp.float32),
            jax.ShapeDtypeStruct((1, 1), jnp.float32),
        ],
        compiler_params=pltpu.CompilerParams(
            dimension_semantics=("arbitrary",),
        ),
    )(codebook, cb2, csq, z)

    commit_loss = (2.0 / (b * d)) * s1[0, 0]
    mean_dist = s2[0, 0] / b
    return zq, idx, commit_loss, mean_dist


# speedup vs baseline: 44.4973x; 10.3970x over previous
import jax
import jax.numpy as jnp
from jax import lax
from jax.experimental import pallas as pl
from jax.experimental.pallas import tpu as pltpu

_K = 16
_D = 5


def _vq_body(cbt_ref, cb2_ref, csq_ref, zt_ref, zqt_ref, idx_ref,
             s1_ref, s2_ref):
    i = pl.program_id(0)
    zt = zt_ref[...]
    blk = zt.shape[1]

    crossT = jax.lax.dot_general(
        cb2_ref[...], zt, (((1,), (0,)), ((), ())),
        preferred_element_type=jnp.float32)
    scoreT = crossT + csq_ref[...]

    zz = zt * zt
    zsqT = jax.lax.dot_general(
        jnp.ones((1, _D), jnp.float32), zz, (((1,), (0,)), ((), ())),
        preferred_element_type=jnp.float32)

    mnT = jnp.min(scoreT, axis=0, keepdims=True)
    iiT = lax.broadcasted_iota(jnp.int32, scoreT.shape, 0)
    idxT = jnp.min(jnp.where(scoreT == mnT, iiT, _K), axis=0, keepdims=True)
    ohT = (iiT == idxT).astype(jnp.float32)

    codesT = jax.lax.dot_general(
        cbt_ref[...], ohT, (((1,), (0,)), ((), ())),
        preferred_element_type=jnp.float32)
    zqt_ref[...] = zt + (codesT - zt)
    idx_ref[...] = idxT.reshape(blk)

    min_d2 = jnp.maximum(mnT + zsqT, 0.0)
    s1 = jnp.sum(min_d2, keepdims=True).reshape(1, 1)
    s2 = jnp.sum(jnp.sqrt(min_d2), keepdims=True).reshape(1, 1)

    @pl.when(i == 0)
    def _init():
        s1_ref[...] = jnp.zeros((1, 1), jnp.float32)
        s2_ref[...] = jnp.zeros((1, 1), jnp.float32)

    s1_ref[...] += s1
    s2_ref[...] += s2


@jax.jit
def kernel(neuromod_state, codebook):
    z = neuromod_state
    if z.ndim == 1:
        z = z[None, :]
    b, d = z.shape
    blk = 16384
    while b % blk != 0:
        blk //= 2
    grid = (b // blk,)

    zt = z.T
    cbt = codebook.T
    cb2 = -2.0 * codebook
    csq = jnp.sum(codebook * codebook, axis=1, keepdims=True)

    zqt, idx, s1, s2 = pl.pallas_call(
        _vq_body,
        grid=grid,
        in_specs=[
            pl.BlockSpec((d, _K), lambda i: (0, 0)),
            pl.BlockSpec((_K, d), lambda i: (0, 0)),
            pl.BlockSpec((_K, 1), lambda i: (0, 0)),
            pl.BlockSpec((d, blk), lambda i: (0, i)),
        ],
        out_specs=[
            pl.BlockSpec((d, blk), lambda i: (0, i)),
            pl.BlockSpec((blk,), lambda i: (i,)),
            pl.BlockSpec((1, 1), lambda i: (0, 0)),
            pl.BlockSpec((1, 1), lambda i: (0, 0)),
        ],
        out_shape=[
            jax.ShapeDtypeStruct((d, b), jnp.float32),
            jax.ShapeDtypeStruct((b,), jnp.int32),
            jax.ShapeDtypeStruct((1, 1), jnp.float32),
            jax.ShapeDtypeStruct((1, 1), jnp.float32),
        ],
        compiler_params=pltpu.CompilerParams(
            dimension_semantics=("arbitrary",),
        ),
    )(cbt, cb2, csq, zt)

    commit_loss = (2.0 / (b * d)) * s1[0, 0]
    mean_dist = s2[0, 0] / b
    return zqt.T, idx, commit_loss, mean_dist


# speedup vs baseline: 55.7907x; 1.2538x over previous
import jax
import jax.numpy as jnp
from jax import lax
from jax.experimental import pallas as pl
from jax.experimental.pallas import tpu as pltpu

_K = 16
_D = 5


def _vq_body(cbt_ref, cb2_ref, csq_ref, zt_ref, zqt_ref, idx_ref,
             s1_ref, s2_ref, a1_ref, a2_ref):
    i = pl.program_id(0)
    n = pl.num_programs(0)
    zt = zt_ref[...]
    blk = zt.shape[1]

    crossT = jax.lax.dot_general(
        cb2_ref[...], zt, (((1,), (0,)), ((), ())),
        preferred_element_type=jnp.float32)
    scoreT = crossT + csq_ref[...]

    zz = zt * zt
    zsqT = jax.lax.dot_general(
        jnp.ones((1, _D), jnp.float32), zz, (((1,), (0,)), ((), ())),
        preferred_element_type=jnp.float32)

    mnT = jnp.min(scoreT, axis=0, keepdims=True)
    iiT = lax.broadcasted_iota(jnp.int32, scoreT.shape, 0)
    idxT = jnp.min(jnp.where(scoreT == mnT, iiT, _K), axis=0, keepdims=True)
    ohT = (iiT == idxT).astype(jnp.float32)

    codesT = jax.lax.dot_general(
        cbt_ref[...], ohT, (((1,), (0,)), ((), ())),
        preferred_element_type=jnp.float32)
    zqt_ref[...] = codesT
    idx_ref[...] = idxT.reshape(blk)

    min_d2 = jnp.maximum(mnT + zsqT, 0.0)

    @pl.when(i == 0)
    def _init():
        a1_ref[...] = jnp.zeros_like(a1_ref)
        a2_ref[...] = jnp.zeros_like(a2_ref)

    a1_ref[...] += min_d2
    a2_ref[...] += jnp.sqrt(min_d2)

    @pl.when(i == n - 1)
    def _fini():
        s1_ref[...] = jnp.sum(a1_ref[...], keepdims=True).reshape(1, 1)
        s2_ref[...] = jnp.sum(a2_ref[...], keepdims=True).reshape(1, 1)


@jax.jit
def kernel(neuromod_state, codebook):
    z = neuromod_state
    if z.ndim == 1:
        z = z[None, :]
    b, d = z.shape
    blk = 32768
    while b % blk != 0:
        blk //= 2
    grid = (b // blk,)

    zt = z.T
    cbt = codebook.T
    cb2 = -2.0 * codebook
    csq = jnp.sum(codebook * codebook, axis=1, keepdims=True)

    zqt, idx, s1, s2 = pl.pallas_call(
        _vq_body,
        grid=grid,
        in_specs=[
            pl.BlockSpec((d, _K), lambda i: (0, 0)),
            pl.BlockSpec((_K, d), lambda i: (0, 0)),
            pl.BlockSpec((_K, 1), lambda i: (0, 0)),
            pl.BlockSpec((d, blk), lambda i: (0, i)),
        ],
        out_specs=[
            pl.BlockSpec((d, blk), lambda i: (0, i)),
            pl.BlockSpec((blk,), lambda i: (i,)),
            pl.BlockSpec((1, 1), lambda i: (0, 0)),
            pl.BlockSpec((1, 1), lambda i: (0, 0)),
        ],
        out_shape=[
            jax.ShapeDtypeStruct((d, b), jnp.float32),
            jax.ShapeDtypeStruct((b,), jnp.int32),
            jax.ShapeDtypeStruct((1, 1), jnp.float32),
            jax.ShapeDtypeStruct((1, 1), jnp.float32),
        ],
        scratch_shapes=[
            pltpu.VMEM((1, blk), jnp.float32),
            pltpu.VMEM((1, blk), jnp.float32),
        ],
        compiler_params=pltpu.CompilerParams(
            dimension_semantics=("arbitrary",),
        ),
    )(cbt, cb2, csq, zt)

    commit_loss = (2.0 / (b * d)) * s1[0, 0]
    mean_dist = s2[0, 0] / b
    return zqt.T, idx, commit_loss, mean_dist
